# baseline (speedup 1.0000x reference)
import jax
import jax.numpy as jnp
from jax import lax
from jax.experimental import pallas as pl
from jax.experimental.pallas import tpu as pltpu
from jax.experimental.pallas import tpu_sc as plsc

_N = 10000
_E = 320000
_T = 160000
_F = 128
_HF = 64

_NC, _NS = 2, 16
_NW = _NC * _NS
_CH = 128

_N_PAD = 10240
_E_PAD = 327680
_T_PAD = 163840
_ECH_W = _E_PAD // _NW // _CH
_TCH_W = _T_PAD // _NW // _CH
_ECH_S = _E_PAD // _NS // _CH
_TCH_S = _T_PAD // _NS // _CH


def _node_body(x_ref, w_ref, b_ref, pw_ref, xb_ref, h0_ref):
    x = x_ref[...]
    xb = jnp.where(x != 0.0, 1.0, 0.0).astype(jnp.float32)
    xb_ref[...] = xb
    y = jnp.dot(xb, w_ref[...], preferred_element_type=jnp.float32) + b_ref[...]
    h0_ref[...] = jnp.where(y >= 0.0, y, pw_ref[0, 0] * y)


def _edge_mm_body(x_ref, w_ref, b_ref, c_ref, pw_ref, o_ref):
    y = jnp.dot(x_ref[...], w_ref[...], preferred_element_type=jnp.float32) + b_ref[...]
    o_ref[...] = jnp.where(y >= 0.0, y, pw_ref[0, 0] * y) + c_ref[...]


def _tri_mm_body(x_ref, w_ref, b_ref, w2_ref, pw_ref, o_ref):
    y = jnp.dot(x_ref[...], w_ref[...], preferred_element_type=jnp.float32) + b_ref[...]
    h = jnp.where(y >= 0.0, y, pw_ref[0, 0] * y)
    o_ref[...] = jnp.dot(h, w2_ref[...], preferred_element_type=jnp.float32)


def _combine_body(h0_ref, z0_ref, z1_ref, o_ref):
    o_ref[...] = (h0_ref[...] + z0_ref[...] + z1_ref[...]) * jnp.float32(1.0 / 3.0)


_BROWS = 512


def _row_spec():
    return pl.BlockSpec((_BROWS, _F), lambda i: (i, 0))


def _half_spec():
    return pl.BlockSpec((_BROWS, _HF), lambda i: (i, 0))


def _full_spec(shape):
    return pl.BlockSpec(shape, lambda i: tuple(0 for _ in shape))


def _node_kernel(X0, W_n, b_n, pw):
    return pl.pallas_call(
        _node_body,
        grid=(pl.cdiv(_N, _BROWS),),
        in_specs=[_row_spec(), _full_spec((_F, _F)), _full_spec((1, _F)),
                  _full_spec((1, 1))],
        out_specs=[_row_spec(), _row_spec()],
        out_shape=[jax.ShapeDtypeStruct((_N, _F), jnp.float32)] * 2,
    )(X0, W_n, b_n.reshape(1, _F), pw.reshape(1, 1))


def _edge_mm_kernel(X1, W_e, b_e, b_tri, pw):
    return pl.pallas_call(
        _edge_mm_body,
        grid=(_E_PAD // _BROWS,),
        in_specs=[_row_spec(), _full_spec((_F, _F)), _full_spec((1, _F)),
                  _full_spec((1, _F)), _full_spec((1, 1))],
        out_specs=_row_spec(),
        out_shape=jax.ShapeDtypeStruct((_E_PAD, _F), jnp.float32),
    )(X1, W_e, b_e.reshape(1, _F), b_tri.reshape(1, _F), pw.reshape(1, 1))


def _tri_mm_kernel(X2, W_t, b_t, W_tri, pw):
    return pl.pallas_call(
        _tri_mm_body,
        grid=(_T_PAD // _BROWS,),
        in_specs=[_row_spec(), _full_spec((_F, _F)), _full_spec((1, _F)),
                  _full_spec((_F, _F)), _full_spec((1, 1))],
        out_specs=_row_spec(),
        out_shape=jax.ShapeDtypeStruct((_T_PAD, _F), jnp.float32),
    )(X2, W_t, b_t.reshape(1, _F), W_tri, pw.reshape(1, 1))


def _combine_kernel(H0, Z0, Z1):
    return pl.pallas_call(
        _combine_body,
        grid=(pl.cdiv(_N, _BROWS),),
        in_specs=[_row_spec(), _row_spec(), _row_spec()],
        out_specs=_row_spec(),
        out_shape=jax.ShapeDtypeStruct((_N, _F), jnp.float32),
    )(H0, Z0, Z1)


def _sc_mesh():
    return plsc.VectorSubcoreMesh(core_axis_name="c", subcore_axis_name="s")


def _wid():
    return lax.axis_index("s") * _NC + lax.axis_index("c")


def _ew_mul(dst, a, b, width=_F):
    def row(r, _):
        for j in range(width // 16):
            sl = pl.ds(j * 16, 16)
            dst[r, sl] = a[r, sl] * b[r, sl]
        return 0
    lax.fori_loop(0, _CH, row, 0)


def _ew_mul_bf(dst, a, b):
    def row(r, _):
        for k in range(_F // 2 // 16):
            sl = pl.ds(k * 16, 16)
            dst[r, sl] = a[r, sl] & b[r, sl]
        return 0
    lax.fori_loop(0, _CH, row, 0)


def _ew_neg(dst, a, width=_HF):
    def row(r, _):
        for j in range(width // 16):
            sl = pl.ds(j * 16, 16)
            dst[r, sl] = -a[r, sl]
        return 0
    lax.fori_loop(0, _CH, row, 0)


def _ew_sub(dst, a, b, width=_HF):
    def row(r, _):
        for j in range(width // 16):
            sl = pl.ds(j * 16, 16)
            dst[r, sl] = a[r, sl] - b[r, sl]
        return 0
    lax.fori_loop(0, _CH, row, 0)


_EC_SLOW, _EC_FAST = 32, 128
_TC_SLOW, _TC_FAST = 16, 64
_SLOW_CORE = 0


def _egather_body(xb, tails2, heads2, x1, IT, IH, A0, B0, A1, B1, s0, s1):
    c = lax.axis_index("c")
    s = lax.axis_index("s")
    slow = c == _SLOW_CORE
    nch = jnp.where(slow, _EC_SLOW, _EC_FAST)
    r0 = jnp.where(slow, s * _EC_SLOW, _NS * _EC_SLOW + s * _EC_FAST)
    pltpu.sync_copy(tails2.at[pl.ds(r0, _EC_FAST)], IT)
    pltpu.sync_copy(heads2.at[pl.ds(r0, _EC_FAST)], IH)

    def body(j, _):
        @pl.when(2 * j + 1 < nch)
        def _():
            ia = 2 * j
            ib = 2 * j + 1
            da1 = pltpu.async_copy(xb.at[IT.at[ia]], A0, s0)
            da2 = pltpu.async_copy(xb.at[IH.at[ia]], B0, s0)
            db1 = pltpu.async_copy(xb.at[IT.at[ib]], A1, s1)
            db2 = pltpu.async_copy(xb.at[IH.at[ib]], B1, s1)
            da1.wait()
            da2.wait()
            _ew_mul(A0, A0, B0)
            pltpu.sync_copy(A0, x1.at[pl.ds((r0 + ia) * _CH, _CH)])
            db1.wait()
            db2.wait()
            _ew_mul(A1, A1, B1)
            pltpu.sync_copy(A1, x1.at[pl.ds((r0 + ib) * _CH, _CH)])
        return 0
    lax.fori_loop(0, _EC_FAST // 2, body, 0)


def _egather_kernel(X0b_pad, tails2, heads2):
    i32, bf16 = jnp.int32, jnp.bfloat16
    fn = pl.kernel(
        _egather_body,
        out_type=jax.ShapeDtypeStruct((_E_PAD, _F), jnp.float32),
        mesh=_sc_mesh(),
        scratch_types=[pltpu.VMEM((_EC_FAST, _CH), i32), pltpu.VMEM((_EC_FAST, _CH), i32),
                       pltpu.VMEM((_CH, _F), jnp.float32), pltpu.VMEM((_CH, _F), jnp.float32),
                       pltpu.VMEM((_CH, _F), jnp.float32), pltpu.VMEM((_CH, _F), jnp.float32),
                       pltpu.SemaphoreType.DMA, pltpu.SemaphoreType.DMA],
    )
    return fn(X0b_pad, tails2, heads2)


def _tgather_body(xb, v02, v12, v22, x2, V0, V1, V2,
                  A0, B0, C0, A1, B1, C1, s0, s1):
    c = lax.axis_index("c")
    s = lax.axis_index("s")
    slow = c == _SLOW_CORE
    nch = jnp.where(slow, _TC_SLOW, _TC_FAST)
    r0 = jnp.where(slow, s * _TC_SLOW, _NS * _TC_SLOW + s * _TC_FAST)
    pltpu.sync_copy(v02.at[pl.ds(r0, _TC_FAST)], V0)
    pltpu.sync_copy(v12.at[pl.ds(r0, _TC_FAST)], V1)
    pltpu.sync_copy(v22.at[pl.ds(r0, _TC_FAST)], V2)

    def body(j, _):
        @pl.when(2 * j + 1 < nch)
        def _():
            ia = 2 * j
            ib = 2 * j + 1
            da1 = pltpu.async_copy(xb.at[V0.at[ia]], A0, s0)
            da2 = pltpu.async_copy(xb.at[V1.at[ia]], B0, s0)
            da3 = pltpu.async_copy(xb.at[V2.at[ia]], C0, s0)
            db1 = pltpu.async_copy(xb.at[V0.at[ib]], A1, s1)
            db2 = pltpu.async_copy(xb.at[V1.at[ib]], B1, s1)
            db3 = pltpu.async_copy(xb.at[V2.at[ib]], C1, s1)
            da1.wait()
            da2.wait()
            da3.wait()
            _ew_mul(A0, A0, B0)
            _ew_mul(A0, A0, C0)
            pltpu.sync_copy(A0, x2.at[pl.ds((r0 + ia) * _CH, _CH)])
            db1.wait()
            db2.wait()
            db3.wait()
            _ew_mul(A1, A1, B1)
            _ew_mul(A1, A1, C1)
            pltpu.sync_copy(A1, x2.at[pl.ds((r0 + ib) * _CH, _CH)])
        return 0
    lax.fori_loop(0, _TC_FAST // 2, body, 0)


def _tgather_kernel(X0b_pad, v02, v12, v22):
    i32, bf16 = jnp.int32, jnp.bfloat16
    fn = pl.kernel(
        _tgather_body,
        out_type=jax.ShapeDtypeStruct((_T_PAD, _F), jnp.float32),
        mesh=_sc_mesh(),
        scratch_types=[pltpu.VMEM((_TC_FAST, _CH), i32)] * 3 +
                      [pltpu.VMEM((_CH, _F), jnp.float32)] * 6 +
                      [pltpu.SemaphoreType.DMA, pltpu.SemaphoreType.DMA],
    )
    return fn(X0b_pad, v02, v12, v22)


def _trans_body(tails, heads, t02, t12, t22,
                p0o, p1o, p2o, n0o, n1o, n2o,
                T0, T1, T2, P0, P1, P2, N0, N1, N2, sm):
    w = _wid()
    r0 = w * _TCH_W
    pltpu.sync_copy(t02.at[pl.ds(r0, _TCH_W)], T0)
    pltpu.sync_copy(t12.at[pl.ds(r0, _TCH_W)], T1)
    pltpu.sync_copy(t22.at[pl.ds(r0, _TCH_W)], T2)

    def body(i, _):
        ds = [pltpu.async_copy(heads.at[T0.at[i]], P0.at[i], sm),
              pltpu.async_copy(tails.at[T0.at[i]], N0.at[i], sm),
              pltpu.async_copy(tails.at[T1.at[i]], P1.at[i], sm),
              pltpu.async_copy(heads.at[T1.at[i]], N1.at[i], sm),
              pltpu.async_copy(heads.at[T2.at[i]], P2.at[i], sm),
              pltpu.async_copy(tails.at[T2.at[i]], N2.at[i], sm)]
        for d in ds:
            d.wait()
        return 0
    lax.fori_loop(0, _TCH_W, body, 0)

    pltpu.sync_copy(P0, p0o.at[pl.ds(r0, _TCH_W)])
    pltpu.sync_copy(P1, p1o.at[pl.ds(r0, _TCH_W)])
    pltpu.sync_copy(P2, p2o.at[pl.ds(r0, _TCH_W)])
    pltpu.sync_copy(N0, n0o.at[pl.ds(r0, _TCH_W)])
    pltpu.sync_copy(N1, n1o.at[pl.ds(r0, _TCH_W)])
    pltpu.sync_copy(N2, n2o.at[pl.ds(r0, _TCH_W)])


def _trans_kernel(tails, heads, t02, t12, t22):
    i32 = jnp.int32
    fn = pl.kernel(
        _trans_body,
        out_type=[jax.ShapeDtypeStruct((_T_PAD // _CH, _CH), i32)] * 6,
        mesh=_sc_mesh(),
        scratch_types=[pltpu.VMEM((_TCH_W, _CH), i32)] * 9 +
                      [pltpu.SemaphoreType.DMA],
    )
    return fn(tails, heads, t02, t12, t22)


_E_PER_W = _E_PAD // _NW
_T_PER_W = _T_PAD // _NW


def _gather_body_r1(xb, tails, heads, v0, v1, v2, t0, t1, t2,
                    x1, x2, p0, p1, p2, n0, n1, n2,
                    ia, ib, ic, A, B, C, sg, sem):
    w = _wid()
    ebase = w * _E_PER_W

    def edge_chunk(i, _):
        base = pl.multiple_of(ebase + i * _CH, _CH)
        pltpu.sync_copy(tails.at[pl.ds(base, _CH)], ia)
        pltpu.sync_copy(heads.at[pl.ds(base, _CH)], ib)
        pltpu.async_copy(xb.at[ia], A, sem).wait()
        pltpu.async_copy(xb.at[ib], B, sem).wait()
        _ew_mul(A, A, B)
        pltpu.sync_copy(A, x1.at[pl.ds(base, _CH)])
        return 0

    lax.fori_loop(0, _E_PER_W // _CH, edge_chunk, 0)

    tbase = w * _T_PER_W

    def tri_chunk(i, _):
        base = pl.multiple_of(tbase + i * _CH, _CH)
        pltpu.sync_copy(v0.at[pl.ds(base, _CH)], ia)
        pltpu.sync_copy(v1.at[pl.ds(base, _CH)], ib)
        pltpu.sync_copy(v2.at[pl.ds(base, _CH)], ic)
        pltpu.async_copy(xb.at[ia], A, sem).wait()
        pltpu.async_copy(xb.at[ib], B, sem).wait()
        pltpu.async_copy(xb.at[ic], C, sem).wait()
        _ew_mul(A, A, B)
        _ew_mul(A, A, C)
        pltpu.sync_copy(A, x2.at[pl.ds(base, _CH)])
        pltpu.sync_copy(t0.at[pl.ds(base, _CH)], ia)
        pltpu.async_copy(heads.at[ia], sg, sem).wait()
        pltpu.sync_copy(sg, p0.at[pl.ds(base, _CH)])
        pltpu.async_copy(tails.at[ia], sg, sem).wait()
        pltpu.sync_copy(sg, n0.at[pl.ds(base, _CH)])
        pltpu.sync_copy(t1.at[pl.ds(base, _CH)], ia)
        pltpu.async_copy(tails.at[ia], sg, sem).wait()
        pltpu.sync_copy(sg, p1.at[pl.ds(base, _CH)])
        pltpu.async_copy(heads.at[ia], sg, sem).wait()
        pltpu.sync_copy(sg, n1.at[pl.ds(base, _CH)])
        pltpu.sync_copy(t2.at[pl.ds(base, _CH)], ia)
        pltpu.async_copy(heads.at[ia], sg, sem).wait()
        pltpu.sync_copy(sg, p2.at[pl.ds(base, _CH)])
        pltpu.async_copy(tails.at[ia], sg, sem).wait()
        pltpu.sync_copy(sg, n2.at[pl.ds(base, _CH)])
        return 0

    lax.fori_loop(0, _T_PER_W // _CH, tri_chunk, 0)


def _gather_kernel_r1(X0b_pad, tails, heads, v0, v1, v2, t0, t1, t2):
    i32 = jnp.int32
    f32 = jnp.float32
    fn = pl.kernel(
        _gather_body_r1,
        out_type=[jax.ShapeDtypeStruct((_E_PAD, _F), f32),
                  jax.ShapeDtypeStruct((_T_PAD, _F), f32)] +
                 [jax.ShapeDtypeStruct((_T_PAD,), i32)] * 6,
        mesh=_sc_mesh(),
        scratch_types=[pltpu.VMEM((_CH,), i32), pltpu.VMEM((_CH,), i32),
                       pltpu.VMEM((_CH,), i32),
                       pltpu.VMEM((_CH, _F), f32), pltpu.VMEM((_CH, _F), f32),
                       pltpu.VMEM((_CH, _F), f32),
                       pltpu.VMEM((_CH,), i32),
                       pltpu.SemaphoreType.DMA],
    )
    return fn(X0b_pad, tails, heads, v0, v1, v2, t0, t1, t2)


def _scatter_body(h1, g2, tails1, heads1, p01, p11, p21, n01, n11, n21,
                  z, it, ih, q0, q1, q2, q3, q4, q5,
                  R, Rn, li, lr, acc):
    c = lax.axis_index("c")
    s = lax.axis_index("s")
    w = s * _NC + c
    rows_sub = _N_PAD // _NS
    zr0 = s * rows_sub

    def zfill(j, _):
        R[j // (_F // 16), pl.ds((j % (_F // 16)) * 16, 16)] = jnp.zeros((16,), jnp.float32)
        return 0
    lax.fori_loop(0, _CH * (_F // 16), zfill, 0)

    def zcp(k, _):
        off = zr0 + k * _CH
        pltpu.sync_copy(R, acc.at[pl.ds(off, _CH)])
        return 0
    lax.fori_loop(0, rows_sub // _CH, zcp, 0)

    plsc.subcore_barrier()

    ebase = w * _ECH_W

    def edge_body(k, _):
        base = (ebase + k) * _CH
        d1 = pltpu.async_copy(tails1.at[pl.ds(base, _CH)], it, li)
        d2 = pltpu.async_copy(heads1.at[pl.ds(base, _CH)], ih, li)
        d3 = pltpu.async_copy(h1.at[pl.ds(base, _CH)], R, lr)
        d1.wait()
        d2.wait()
        d3.wait()
        _ew_neg(Rn, R, _F)
        pltpu.sync_copy(R, acc.at[ih], add=True)
        pltpu.sync_copy(Rn, acc.at[it], add=True)
        return 0

    lax.fori_loop(0, _ECH_W, edge_body, 0)

    tbase = w * _TCH_W

    def tri_body(k, _):
        base = (tbase + k) * _CH
        dd = [pltpu.async_copy(p01.at[pl.ds(base, _CH)], q0, li),
              pltpu.async_copy(p11.at[pl.ds(base, _CH)], q1, li),
              pltpu.async_copy(p21.at[pl.ds(base, _CH)], q2, li),
              pltpu.async_copy(n01.at[pl.ds(base, _CH)], q3, li),
              pltpu.async_copy(n11.at[pl.ds(base, _CH)], q4, li),
              pltpu.async_copy(n21.at[pl.ds(base, _CH)], q5, li),
              pltpu.async_copy(g2.at[pl.ds(base, _CH)], R, lr)]
        for d in dd:
            d.wait()
        _ew_neg(Rn, R, _F)
        pltpu.sync_copy(R, acc.at[q0], add=True)
        pltpu.sync_copy(R, acc.at[q1], add=True)
        pltpu.sync_copy(R, acc.at[q2], add=True)
        pltpu.sync_copy(Rn, acc.at[q3], add=True)
        pltpu.sync_copy(Rn, acc.at[q4], add=True)
        pltpu.sync_copy(Rn, acc.at[q5], add=True)
        return 0

    lax.fori_loop(0, _TCH_W, tri_body, 0)

    plsc.subcore_barrier()

    def outcp(k, _):
        off = zr0 + k * _CH
        pltpu.sync_copy(acc.at[pl.ds(off, _CH)], z.at[c, pl.ds(off, _CH)])
        return 0
    lax.fori_loop(0, rows_sub // _CH, outcp, 0)


def _scatter_kernel(H1p, G2, tails1, heads1, p01, p11, p21, n01, n11, n21):
    i32, f32 = jnp.int32, jnp.float32
    fn = pl.kernel(
        _scatter_body,
        out_type=jax.ShapeDtypeStruct((_NC, _N_PAD, _F), f32),
        mesh=_sc_mesh(),
        scratch_types=[pltpu.VMEM((_CH,), i32)] * 8 +
                      [pltpu.VMEM((_CH, _F), f32)] * 2 +
                      [pltpu.SemaphoreType.DMA] * 2 +
                      [pltpu.VMEM_SHARED((_N_PAD, _F), f32)],
    )
    return fn(H1p, G2, tails1, heads1, p01, p11, p21, n01, n11, n21)


def kernel(X0, edge_index, tri_index, tri_edge_index, W_n, b_n, W_e, b_e,
           W_t, b_t, W_tri, b_tri, prelu_w):
    i32 = jnp.int32
    f32 = jnp.float32

    def padi2(col, total, fill):
        v = jnp.concatenate([col, jnp.full((total - col.shape[0],), fill, i32)])
        return v.reshape(total // _CH, _CH)

    tails2 = padi2(edge_index[:, 0], _E_PAD, _N)
    heads2 = padi2(edge_index[:, 1], _E_PAD, _N)
    v02 = padi2(tri_index[:, 0], _T_PAD, _N)
    v12 = padi2(tri_index[:, 1], _T_PAD, _N)
    v22 = padi2(tri_index[:, 2], _T_PAD, _N)
    t02 = padi2(tri_edge_index[:, 0], _T_PAD, _E)
    t12 = padi2(tri_edge_index[:, 1], _T_PAD, _E)
    t22 = padi2(tri_edge_index[:, 2], _T_PAD, _E)
    tails1 = tails2.reshape(_E_PAD)
    heads1 = heads2.reshape(_E_PAD)

    X0b, H0 = _node_kernel(X0, W_n, b_n, prelu_w)
    X0b_pad = jnp.concatenate([X0b, jnp.zeros((_N_PAD - _N, _F), f32)])

    X1 = _egather_kernel(X0b_pad, tails2, heads2)
    X2 = _tgather_kernel(X0b_pad, v02, v12, v22)
    p02, p12, p22, n02, n12, n22 = _trans_kernel(tails1, heads1, t02, t12, t22)

    H1p = _edge_mm_kernel(X1, W_e, b_e, b_tri, prelu_w)
    G2 = _tri_mm_kernel(X2, W_t, b_t, W_tri, prelu_w)

    Z = _scatter_kernel(H1p, G2, tails1, heads1,
                        p02.reshape(_T_PAD), p12.reshape(_T_PAD),
                        p22.reshape(_T_PAD), n02.reshape(_T_PAD),
                        n12.reshape(_T_PAD), n22.reshape(_T_PAD))

    return _combine_kernel(H0, Z[0, :_N], Z[1, :_N])


# speedup vs baseline: 1.0326x; 1.0326x over previous
import jax
import jax.numpy as jnp
from jax import lax
from jax.experimental import pallas as pl
from jax.experimental.pallas import tpu as pltpu
from jax.experimental.pallas import tpu_sc as plsc

_N = 10000
_E = 320000
_T = 160000
_F = 128
_HF = 64

_NC, _NS = 2, 16
_NW = _NC * _NS
_CH = 128

_N_PAD = 10240
_E_PAD = 327680
_T_PAD = 163840
_ECH_W = _E_PAD // _NW // _CH
_TCH_W = _T_PAD // _NW // _CH
_ECH_S = _E_PAD // _NS // _CH
_TCH_S = _T_PAD // _NS // _CH


def _node_body(x_ref, w_ref, b_ref, pw_ref, xb_ref, h0_ref):
    x = x_ref[...]
    xb = jnp.where(x != 0.0, 1.0, 0.0).astype(jnp.float32)
    xb_ref[...] = xb
    y = jnp.dot(xb, w_ref[...], preferred_element_type=jnp.float32) + b_ref[...]
    h0_ref[...] = jnp.where(y >= 0.0, y, pw_ref[0, 0] * y)


def _edge_mm_body(x_ref, w_ref, b_ref, c_ref, pw_ref, o_ref):
    y = jnp.dot(x_ref[...], w_ref[...], preferred_element_type=jnp.float32) + b_ref[...]
    o_ref[...] = jnp.where(y >= 0.0, y, pw_ref[0, 0] * y) + c_ref[...]


def _tri_mm_body(x_ref, w_ref, b_ref, w2_ref, pw_ref, o_ref):
    y = jnp.dot(x_ref[...], w_ref[...], preferred_element_type=jnp.float32) + b_ref[...]
    h = jnp.where(y >= 0.0, y, pw_ref[0, 0] * y)
    o_ref[...] = jnp.dot(h, w2_ref[...], preferred_element_type=jnp.float32)


def _combine_body(h0_ref, z0_ref, z1_ref, o_ref):
    o_ref[...] = (h0_ref[...] + z0_ref[...] + z1_ref[...]) * jnp.float32(1.0 / 3.0)


_BROWS = 512


def _row_spec():
    return pl.BlockSpec((_BROWS, _F), lambda i: (i, 0))


def _half_spec():
    return pl.BlockSpec((_BROWS, _HF), lambda i: (i, 0))


def _full_spec(shape):
    return pl.BlockSpec(shape, lambda i: tuple(0 for _ in shape))


def _node_kernel(X0, W_n, b_n, pw):
    return pl.pallas_call(
        _node_body,
        grid=(pl.cdiv(_N, _BROWS),),
        in_specs=[_row_spec(), _full_spec((_F, _F)), _full_spec((1, _F)),
                  _full_spec((1, 1))],
        out_specs=[_row_spec(), _row_spec()],
        out_shape=[jax.ShapeDtypeStruct((_N, _F), jnp.float32)] * 2,
    )(X0, W_n, b_n.reshape(1, _F), pw.reshape(1, 1))


def _edge_mm_kernel(X1, W_e, b_e, b_tri, pw):
    return pl.pallas_call(
        _edge_mm_body,
        grid=(_E_PAD // _BROWS,),
        in_specs=[_row_spec(), _full_spec((_F, _F)), _full_spec((1, _F)),
                  _full_spec((1, _F)), _full_spec((1, 1))],
        out_specs=_row_spec(),
        out_shape=jax.ShapeDtypeStruct((_E_PAD, _F), jnp.float32),
    )(X1, W_e, b_e.reshape(1, _F), b_tri.reshape(1, _F), pw.reshape(1, 1))


def _tri_mm_kernel(X2, W_t, b_t, W_tri, pw):
    return pl.pallas_call(
        _tri_mm_body,
        grid=(_T_PAD // _BROWS,),
        in_specs=[_row_spec(), _full_spec((_F, _F)), _full_spec((1, _F)),
                  _full_spec((_F, _F)), _full_spec((1, 1))],
        out_specs=_row_spec(),
        out_shape=jax.ShapeDtypeStruct((_T_PAD, _F), jnp.float32),
    )(X2, W_t, b_t.reshape(1, _F), W_tri, pw.reshape(1, 1))


def _combine_kernel(H0, Z0, Z1):
    return pl.pallas_call(
        _combine_body,
        grid=(pl.cdiv(_N, _BROWS),),
        in_specs=[_row_spec(), _row_spec(), _row_spec()],
        out_specs=_row_spec(),
        out_shape=jax.ShapeDtypeStruct((_N, _F), jnp.float32),
    )(H0, Z0, Z1)


def _sc_mesh():
    return plsc.VectorSubcoreMesh(core_axis_name="c", subcore_axis_name="s")


def _wid():
    return lax.axis_index("s") * _NC + lax.axis_index("c")


def _ew_mul(dst, a, b, width=_F):
    def row(r, _):
        for j in range(width // 16):
            sl = pl.ds(j * 16, 16)
            dst[r, sl] = a[r, sl] * b[r, sl]
        return 0
    lax.fori_loop(0, _CH, row, 0)


def _ew_mul_bf(dst, a, b):
    def row(r, _):
        for k in range(_F // 2 // 16):
            sl = pl.ds(k * 16, 16)
            dst[r, sl] = a[r, sl] & b[r, sl]
        return 0
    lax.fori_loop(0, _CH, row, 0)


def _ew_neg(dst, a, width=_HF):
    def row(r, _):
        for j in range(width // 16):
            sl = pl.ds(j * 16, 16)
            dst[r, sl] = -a[r, sl]
        return 0
    lax.fori_loop(0, _CH, row, 0)


def _ew_sub(dst, a, b, width=_HF):
    def row(r, _):
        for j in range(width // 16):
            sl = pl.ds(j * 16, 16)
            dst[r, sl] = a[r, sl] - b[r, sl]
        return 0
    lax.fori_loop(0, _CH, row, 0)


_EC_SLOW, _EC_FAST = 32, 128
_TC_SLOW, _TC_FAST = 16, 64
_SLOW_CORE = 1


def _egather_body(xb, tails2, heads2, x1, IT, IH, A0, B0, A1, B1, s0, s1):
    c = lax.axis_index("c")
    s = lax.axis_index("s")
    slow = c == _SLOW_CORE
    nch = jnp.where(slow, _EC_SLOW, _EC_FAST)
    r0 = jnp.where(slow, s * _EC_SLOW, _NS * _EC_SLOW + s * _EC_FAST)
    pltpu.sync_copy(tails2.at[pl.ds(r0, _EC_FAST)], IT)
    pltpu.sync_copy(heads2.at[pl.ds(r0, _EC_FAST)], IH)

    def body(j, _):
        @pl.when(2 * j + 1 < nch)
        def _():
            ia = 2 * j
            ib = 2 * j + 1
            da1 = pltpu.async_copy(xb.at[IT.at[ia]], A0, s0)
            da2 = pltpu.async_copy(xb.at[IH.at[ia]], B0, s0)
            db1 = pltpu.async_copy(xb.at[IT.at[ib]], A1, s1)
            db2 = pltpu.async_copy(xb.at[IH.at[ib]], B1, s1)
            da1.wait()
            da2.wait()
            _ew_mul(A0, A0, B0)
            pltpu.sync_copy(A0, x1.at[pl.ds((r0 + ia) * _CH, _CH)])
            db1.wait()
            db2.wait()
            _ew_mul(A1, A1, B1)
            pltpu.sync_copy(A1, x1.at[pl.ds((r0 + ib) * _CH, _CH)])
        return 0
    lax.fori_loop(0, _EC_FAST // 2, body, 0)


def _egather_kernel(X0b_pad, tails2, heads2):
    i32, bf16 = jnp.int32, jnp.bfloat16
    fn = pl.kernel(
        _egather_body,
        out_type=jax.ShapeDtypeStruct((_E_PAD, _F), jnp.float32),
        mesh=_sc_mesh(),
        scratch_types=[pltpu.VMEM((_EC_FAST, _CH), i32), pltpu.VMEM((_EC_FAST, _CH), i32),
                       pltpu.VMEM((_CH, _F), jnp.float32), pltpu.VMEM((_CH, _F), jnp.float32),
                       pltpu.VMEM((_CH, _F), jnp.float32), pltpu.VMEM((_CH, _F), jnp.float32),
                       pltpu.SemaphoreType.DMA, pltpu.SemaphoreType.DMA],
    )
    return fn(X0b_pad, tails2, heads2)


def _tgather_body(xb, v02, v12, v22, x2, V0, V1, V2,
                  A0, B0, C0, A1, B1, C1, s0, s1):
    c = lax.axis_index("c")
    s = lax.axis_index("s")
    slow = c == _SLOW_CORE
    nch = jnp.where(slow, _TC_SLOW, _TC_FAST)
    r0 = jnp.where(slow, s * _TC_SLOW, _NS * _TC_SLOW + s * _TC_FAST)
    pltpu.sync_copy(v02.at[pl.ds(r0, _TC_FAST)], V0)
    pltpu.sync_copy(v12.at[pl.ds(r0, _TC_FAST)], V1)
    pltpu.sync_copy(v22.at[pl.ds(r0, _TC_FAST)], V2)

    def body(j, _):
        @pl.when(2 * j + 1 < nch)
        def _():
            ia = 2 * j
            ib = 2 * j + 1
            da1 = pltpu.async_copy(xb.at[V0.at[ia]], A0, s0)
            da2 = pltpu.async_copy(xb.at[V1.at[ia]], B0, s0)
            da3 = pltpu.async_copy(xb.at[V2.at[ia]], C0, s0)
            db1 = pltpu.async_copy(xb.at[V0.at[ib]], A1, s1)
            db2 = pltpu.async_copy(xb.at[V1.at[ib]], B1, s1)
            db3 = pltpu.async_copy(xb.at[V2.at[ib]], C1, s1)
            da1.wait()
            da2.wait()
            da3.wait()
            _ew_mul(A0, A0, B0)
            _ew_mul(A0, A0, C0)
            pltpu.sync_copy(A0, x2.at[pl.ds((r0 + ia) * _CH, _CH)])
            db1.wait()
            db2.wait()
            db3.wait()
            _ew_mul(A1, A1, B1)
            _ew_mul(A1, A1, C1)
            pltpu.sync_copy(A1, x2.at[pl.ds((r0 + ib) * _CH, _CH)])
        return 0
    lax.fori_loop(0, _TC_FAST // 2, body, 0)


def _tgather_kernel(X0b_pad, v02, v12, v22):
    i32, bf16 = jnp.int32, jnp.bfloat16
    fn = pl.kernel(
        _tgather_body,
        out_type=jax.ShapeDtypeStruct((_T_PAD, _F), jnp.float32),
        mesh=_sc_mesh(),
        scratch_types=[pltpu.VMEM((_TC_FAST, _CH), i32)] * 3 +
                      [pltpu.VMEM((_CH, _F), jnp.float32)] * 6 +
                      [pltpu.SemaphoreType.DMA, pltpu.SemaphoreType.DMA],
    )
    return fn(X0b_pad, v02, v12, v22)


def _trans_body(tails, heads, t02, t12, t22,
                p0o, p1o, p2o, n0o, n1o, n2o,
                T0, T1, T2, P0, P1, P2, N0, N1, N2, sm):
    w = _wid()
    r0 = w * _TCH_W
    pltpu.sync_copy(t02.at[pl.ds(r0, _TCH_W)], T0)
    pltpu.sync_copy(t12.at[pl.ds(r0, _TCH_W)], T1)
    pltpu.sync_copy(t22.at[pl.ds(r0, _TCH_W)], T2)

    def body(i, _):
        ds = [pltpu.async_copy(heads.at[T0.at[i]], P0.at[i], sm),
              pltpu.async_copy(tails.at[T0.at[i]], N0.at[i], sm),
              pltpu.async_copy(tails.at[T1.at[i]], P1.at[i], sm),
              pltpu.async_copy(heads.at[T1.at[i]], N1.at[i], sm),
              pltpu.async_copy(heads.at[T2.at[i]], P2.at[i], sm),
              pltpu.async_copy(tails.at[T2.at[i]], N2.at[i], sm)]
        for d in ds:
            d.wait()
        return 0
    lax.fori_loop(0, _TCH_W, body, 0)

    pltpu.sync_copy(P0, p0o.at[pl.ds(r0, _TCH_W)])
    pltpu.sync_copy(P1, p1o.at[pl.ds(r0, _TCH_W)])
    pltpu.sync_copy(P2, p2o.at[pl.ds(r0, _TCH_W)])
    pltpu.sync_copy(N0, n0o.at[pl.ds(r0, _TCH_W)])
    pltpu.sync_copy(N1, n1o.at[pl.ds(r0, _TCH_W)])
    pltpu.sync_copy(N2, n2o.at[pl.ds(r0, _TCH_W)])


def _trans_kernel(tails, heads, t02, t12, t22):
    i32 = jnp.int32
    fn = pl.kernel(
        _trans_body,
        out_type=[jax.ShapeDtypeStruct((_T_PAD // _CH, _CH), i32)] * 6,
        mesh=_sc_mesh(),
        scratch_types=[pltpu.VMEM((_TCH_W, _CH), i32)] * 9 +
                      [pltpu.SemaphoreType.DMA],
    )
    return fn(tails, heads, t02, t12, t22)


_E_PER_W = _E_PAD // _NW
_T_PER_W = _T_PAD // _NW


def _gather_body_r1(xb, tails, heads, v0, v1, v2, t0, t1, t2,
                    x1, x2, p0, p1, p2, n0, n1, n2,
                    ia, ib, ic, A, B, C, sg, sem):
    w = _wid()
    ebase = w * _E_PER_W

    def edge_chunk(i, _):
        base = pl.multiple_of(ebase + i * _CH, _CH)
        pltpu.sync_copy(tails.at[pl.ds(base, _CH)], ia)
        pltpu.sync_copy(heads.at[pl.ds(base, _CH)], ib)
        pltpu.async_copy(xb.at[ia], A, sem).wait()
        pltpu.async_copy(xb.at[ib], B, sem).wait()
        _ew_mul(A, A, B)
        pltpu.sync_copy(A, x1.at[pl.ds(base, _CH)])
        return 0

    lax.fori_loop(0, _E_PER_W // _CH, edge_chunk, 0)

    tbase = w * _T_PER_W

    def tri_chunk(i, _):
        base = pl.multiple_of(tbase + i * _CH, _CH)
        pltpu.sync_copy(v0.at[pl.ds(base, _CH)], ia)
        pltpu.sync_copy(v1.at[pl.ds(base, _CH)], ib)
        pltpu.sync_copy(v2.at[pl.ds(base, _CH)], ic)
        pltpu.async_copy(xb.at[ia], A, sem).wait()
        pltpu.async_copy(xb.at[ib], B, sem).wait()
        pltpu.async_copy(xb.at[ic], C, sem).wait()
        _ew_mul(A, A, B)
        _ew_mul(A, A, C)
        pltpu.sync_copy(A, x2.at[pl.ds(base, _CH)])
        pltpu.sync_copy(t0.at[pl.ds(base, _CH)], ia)
        pltpu.async_copy(heads.at[ia], sg, sem).wait()
        pltpu.sync_copy(sg, p0.at[pl.ds(base, _CH)])
        pltpu.async_copy(tails.at[ia], sg, sem).wait()
        pltpu.sync_copy(sg, n0.at[pl.ds(base, _CH)])
        pltpu.sync_copy(t1.at[pl.ds(base, _CH)], ia)
        pltpu.async_copy(tails.at[ia], sg, sem).wait()
        pltpu.sync_copy(sg, p1.at[pl.ds(base, _CH)])
        pltpu.async_copy(heads.at[ia], sg, sem).wait()
        pltpu.sync_copy(sg, n1.at[pl.ds(base, _CH)])
        pltpu.sync_copy(t2.at[pl.ds(base, _CH)], ia)
        pltpu.async_copy(heads.at[ia], sg, sem).wait()
        pltpu.sync_copy(sg, p2.at[pl.ds(base, _CH)])
        pltpu.async_copy(tails.at[ia], sg, sem).wait()
        pltpu.sync_copy(sg, n2.at[pl.ds(base, _CH)])
        return 0

    lax.fori_loop(0, _T_PER_W // _CH, tri_chunk, 0)


def _gather_kernel_r1(X0b_pad, tails, heads, v0, v1, v2, t0, t1, t2):
    i32 = jnp.int32
    f32 = jnp.float32
    fn = pl.kernel(
        _gather_body_r1,
        out_type=[jax.ShapeDtypeStruct((_E_PAD, _F), f32),
                  jax.ShapeDtypeStruct((_T_PAD, _F), f32)] +
                 [jax.ShapeDtypeStruct((_T_PAD,), i32)] * 6,
        mesh=_sc_mesh(),
        scratch_types=[pltpu.VMEM((_CH,), i32), pltpu.VMEM((_CH,), i32),
                       pltpu.VMEM((_CH,), i32),
                       pltpu.VMEM((_CH, _F), f32), pltpu.VMEM((_CH, _F), f32),
                       pltpu.VMEM((_CH, _F), f32),
                       pltpu.VMEM((_CH,), i32),
                       pltpu.SemaphoreType.DMA],
    )
    return fn(X0b_pad, tails, heads, v0, v1, v2, t0, t1, t2)


def _scatter_body(h1, g2, tails1, heads1, p01, p11, p21, n01, n11, n21,
                  z, it, ih, q0, q1, q2, q3, q4, q5,
                  R, Rn, li, lr, acc):
    c = lax.axis_index("c")
    s = lax.axis_index("s")
    w = s * _NC + c
    rows_sub = _N_PAD // _NS
    zr0 = s * rows_sub

    def zfill(j, _):
        R[j // (_F // 16), pl.ds((j % (_F // 16)) * 16, 16)] = jnp.zeros((16,), jnp.float32)
        return 0
    lax.fori_loop(0, _CH * (_F // 16), zfill, 0)

    def zcp(k, _):
        off = zr0 + k * _CH
        pltpu.sync_copy(R, acc.at[pl.ds(off, _CH)])
        return 0
    lax.fori_loop(0, rows_sub // _CH, zcp, 0)

    plsc.subcore_barrier()

    ebase = w * _ECH_W

    def edge_body(k, _):
        base = (ebase + k) * _CH
        d1 = pltpu.async_copy(tails1.at[pl.ds(base, _CH)], it, li)
        d2 = pltpu.async_copy(heads1.at[pl.ds(base, _CH)], ih, li)
        d3 = pltpu.async_copy(h1.at[pl.ds(base, _CH)], R, lr)
        d1.wait()
        d2.wait()
        d3.wait()
        _ew_neg(Rn, R, _F)
        pltpu.sync_copy(R, acc.at[ih], add=True)
        pltpu.sync_copy(Rn, acc.at[it], add=True)
        return 0

    lax.fori_loop(0, _ECH_W, edge_body, 0)

    tbase = w * _TCH_W

    def tri_body(k, _):
        base = (tbase + k) * _CH
        dd = [pltpu.async_copy(p01.at[pl.ds(base, _CH)], q0, li),
              pltpu.async_copy(p11.at[pl.ds(base, _CH)], q1, li),
              pltpu.async_copy(p21.at[pl.ds(base, _CH)], q2, li),
              pltpu.async_copy(n01.at[pl.ds(base, _CH)], q3, li),
              pltpu.async_copy(n11.at[pl.ds(base, _CH)], q4, li),
              pltpu.async_copy(n21.at[pl.ds(base, _CH)], q5, li),
              pltpu.async_copy(g2.at[pl.ds(base, _CH)], R, lr)]
        for d in dd:
            d.wait()
        _ew_neg(Rn, R, _F)
        pltpu.sync_copy(R, acc.at[q0], add=True)
        pltpu.sync_copy(R, acc.at[q1], add=True)
        pltpu.sync_copy(R, acc.at[q2], add=True)
        pltpu.sync_copy(Rn, acc.at[q3], add=True)
        pltpu.sync_copy(Rn, acc.at[q4], add=True)
        pltpu.sync_copy(Rn, acc.at[q5], add=True)
        return 0

    lax.fori_loop(0, _TCH_W, tri_body, 0)

    plsc.subcore_barrier()

    def outcp(k, _):
        off = zr0 + k * _CH
        pltpu.sync_copy(acc.at[pl.ds(off, _CH)], z.at[c, pl.ds(off, _CH)])
        return 0
    lax.fori_loop(0, rows_sub // _CH, outcp, 0)


def _scatter_kernel(H1p, G2, tails1, heads1, p01, p11, p21, n01, n11, n21):
    i32, f32 = jnp.int32, jnp.float32
    fn = pl.kernel(
        _scatter_body,
        out_type=jax.ShapeDtypeStruct((_NC, _N_PAD, _F), f32),
        mesh=_sc_mesh(),
        scratch_types=[pltpu.VMEM((_CH,), i32)] * 8 +
                      [pltpu.VMEM((_CH, _F), f32)] * 2 +
                      [pltpu.SemaphoreType.DMA] * 2 +
                      [pltpu.VMEM_SHARED((_N_PAD, _F), f32)],
    )
    return fn(H1p, G2, tails1, heads1, p01, p11, p21, n01, n11, n21)


def kernel(X0, edge_index, tri_index, tri_edge_index, W_n, b_n, W_e, b_e,
           W_t, b_t, W_tri, b_tri, prelu_w):
    i32 = jnp.int32
    f32 = jnp.float32

    def padi2(col, total, fill):
        v = jnp.concatenate([col, jnp.full((total - col.shape[0],), fill, i32)])
        return v.reshape(total // _CH, _CH)

    tails2 = padi2(edge_index[:, 0], _E_PAD, _N)
    heads2 = padi2(edge_index[:, 1], _E_PAD, _N)
    v02 = padi2(tri_index[:, 0], _T_PAD, _N)
    v12 = padi2(tri_index[:, 1], _T_PAD, _N)
    v22 = padi2(tri_index[:, 2], _T_PAD, _N)
    t02 = padi2(tri_edge_index[:, 0], _T_PAD, _E)
    t12 = padi2(tri_edge_index[:, 1], _T_PAD, _E)
    t22 = padi2(tri_edge_index[:, 2], _T_PAD, _E)
    tails1 = tails2.reshape(_E_PAD)
    heads1 = heads2.reshape(_E_PAD)

    X0b, H0 = _node_kernel(X0, W_n, b_n, prelu_w)
    X0b_pad = jnp.concatenate([X0b, jnp.zeros((_N_PAD - _N, _F), f32)])

    X1 = _egather_kernel(X0b_pad, tails2, heads2)
    X2 = _tgather_kernel(X0b_pad, v02, v12, v22)
    p02, p12, p22, n02, n12, n22 = _trans_kernel(tails1, heads1, t02, t12, t22)

    H1p = _edge_mm_kernel(X1, W_e, b_e, b_tri, prelu_w)
    G2 = _tri_mm_kernel(X2, W_t, b_t, W_tri, prelu_w)

    Z = _scatter_kernel(H1p, G2, tails1, heads1,
                        p02.reshape(_T_PAD), p12.reshape(_T_PAD),
                        p22.reshape(_T_PAD), n02.reshape(_T_PAD),
                        n12.reshape(_T_PAD), n22.reshape(_T_PAD))

    return _combine_kernel(H0, Z[0, :_N], Z[1, :_N])


# speedup vs baseline: 1.9456x; 1.8842x over previous
import jax
import jax.numpy as jnp
from jax import lax
from jax.experimental import pallas as pl
from jax.experimental.pallas import tpu as pltpu
from jax.experimental.pallas import tpu_sc as plsc

_N = 10000
_E = 320000
_T = 160000
_F = 128
_HF = 64

_NC, _NS = 2, 16
_NW = _NC * _NS
_CH = 128

_N_PAD = 10240
_E_PAD = 327680
_T_PAD = 163840
_ECH_W = _E_PAD // _NW // _CH
_TCH_W = _T_PAD // _NW // _CH
_ECH_S = _E_PAD // _NS // _CH
_TCH_S = _T_PAD // _NS // _CH


def _node_body(x_ref, w_ref, b_ref, pw_ref, xb_ref, h0_ref):
    x = x_ref[...]
    xb = jnp.where(x != 0.0, 1.0, 0.0).astype(jnp.float32)
    xb_ref[...] = xb
    y = jnp.dot(xb, w_ref[...], preferred_element_type=jnp.float32) + b_ref[...]
    h0_ref[...] = jnp.where(y >= 0.0, y, pw_ref[0, 0] * y)


def _edge_mm_body(x_ref, w_ref, b_ref, c_ref, pw_ref, o_ref):
    y = jnp.dot(x_ref[...], w_ref[...], preferred_element_type=jnp.float32) + b_ref[...]
    o_ref[...] = jnp.where(y >= 0.0, y, pw_ref[0, 0] * y) + c_ref[...]


def _tri_mm_body(x_ref, w_ref, b_ref, w2_ref, pw_ref, o_ref):
    y = jnp.dot(x_ref[...], w_ref[...], preferred_element_type=jnp.float32) + b_ref[...]
    h = jnp.where(y >= 0.0, y, pw_ref[0, 0] * y)
    o_ref[...] = jnp.dot(h, w2_ref[...], preferred_element_type=jnp.float32)


def _combine_body(h0_ref, z0_ref, z1_ref, o_ref):
    o_ref[...] = (h0_ref[...] + z0_ref[...] + z1_ref[...]) * jnp.float32(1.0 / 3.0)


_BROWS = 512


def _row_spec():
    return pl.BlockSpec((_BROWS, _F), lambda i: (i, 0))


def _half_spec():
    return pl.BlockSpec((_BROWS, _HF), lambda i: (i, 0))


def _full_spec(shape):
    return pl.BlockSpec(shape, lambda i: tuple(0 for _ in shape))


def _node_kernel(X0, W_n, b_n, pw):
    return pl.pallas_call(
        _node_body,
        grid=(pl.cdiv(_N, _BROWS),),
        in_specs=[_row_spec(), _full_spec((_F, _F)), _full_spec((1, _F)),
                  _full_spec((1, 1))],
        out_specs=[_row_spec(), _row_spec()],
        out_shape=[jax.ShapeDtypeStruct((_N, _F), jnp.float32)] * 2,
    )(X0, W_n, b_n.reshape(1, _F), pw.reshape(1, 1))


def _edge_mm_kernel(X1, W_e, b_e, b_tri, pw):
    return pl.pallas_call(
        _edge_mm_body,
        grid=(_E_PAD // _BROWS,),
        in_specs=[_row_spec(), _full_spec((_F, _F)), _full_spec((1, _F)),
                  _full_spec((1, _F)), _full_spec((1, 1))],
        out_specs=_row_spec(),
        out_shape=jax.ShapeDtypeStruct((_E_PAD, _F), jnp.float32),
    )(X1, W_e, b_e.reshape(1, _F), b_tri.reshape(1, _F), pw.reshape(1, 1))


def _tri_mm_kernel(X2, W_t, b_t, W_tri, pw):
    return pl.pallas_call(
        _tri_mm_body,
        grid=(_T_PAD // _BROWS,),
        in_specs=[_row_spec(), _full_spec((_F, _F)), _full_spec((1, _F)),
                  _full_spec((_F, _F)), _full_spec((1, 1))],
        out_specs=_row_spec(),
        out_shape=jax.ShapeDtypeStruct((_T_PAD, _F), jnp.float32),
    )(X2, W_t, b_t.reshape(1, _F), W_tri, pw.reshape(1, 1))


def _combine_kernel(H0, Z0, Z1):
    return pl.pallas_call(
        _combine_body,
        grid=(pl.cdiv(_N, _BROWS),),
        in_specs=[_row_spec(), _row_spec(), _row_spec()],
        out_specs=_row_spec(),
        out_shape=jax.ShapeDtypeStruct((_N, _F), jnp.float32),
    )(H0, Z0, Z1)


def _sc_mesh():
    return plsc.VectorSubcoreMesh(core_axis_name="c", subcore_axis_name="s")


def _wid():
    return lax.axis_index("s") * _NC + lax.axis_index("c")


def _ew_mul(dst, a, b, width=_F):
    def row(r, _):
        for j in range(width // 16):
            sl = pl.ds(j * 16, 16)
            dst[r, sl] = a[r, sl] * b[r, sl]
        return 0
    lax.fori_loop(0, _CH, row, 0)


def _ew_mul_bf(dst, a, b):
    def row(r, _):
        for k in range(_F // 2 // 16):
            sl = pl.ds(k * 16, 16)
            dst[r, sl] = a[r, sl] & b[r, sl]
        return 0
    lax.fori_loop(0, _CH, row, 0)


def _ew_neg(dst, a, width=_HF):
    def row(r, _):
        for j in range(width // 16):
            sl = pl.ds(j * 16, 16)
            dst[r, sl] = -a[r, sl]
        return 0
    lax.fori_loop(0, _CH, row, 0)


def _ew_sub(dst, a, b, width=_HF):
    def row(r, _):
        for j in range(width // 16):
            sl = pl.ds(j * 16, 16)
            dst[r, sl] = a[r, sl] - b[r, sl]
        return 0
    lax.fori_loop(0, _CH, row, 0)


_EC_SLOW, _EC_FAST = 32, 128
_TC_SLOW, _TC_FAST = 16, 64
_SLOW_CORE = 1


def _egather_body(xb, tails2, heads2, x1, IT, IH, A0, B0, A1, B1, s0, s1):
    c = lax.axis_index("c")
    s = lax.axis_index("s")
    slow = c == _SLOW_CORE
    nch = jnp.where(slow, _EC_SLOW, _EC_FAST)
    r0 = jnp.where(slow, s * _EC_SLOW, _NS * _EC_SLOW + s * _EC_FAST)
    pltpu.sync_copy(tails2.at[pl.ds(r0, _EC_FAST)], IT)
    pltpu.sync_copy(heads2.at[pl.ds(r0, _EC_FAST)], IH)

    def body(j, _):
        @pl.when(2 * j + 1 < nch)
        def _():
            ia = 2 * j
            ib = 2 * j + 1
            da1 = pltpu.async_copy(xb.at[IT.at[ia]], A0, s0)
            da2 = pltpu.async_copy(xb.at[IH.at[ia]], B0, s0)
            db1 = pltpu.async_copy(xb.at[IT.at[ib]], A1, s1)
            db2 = pltpu.async_copy(xb.at[IH.at[ib]], B1, s1)
            da1.wait()
            da2.wait()
            _ew_mul(A0, A0, B0)
            pltpu.sync_copy(A0, x1.at[pl.ds((r0 + ia) * _CH, _CH)])
            db1.wait()
            db2.wait()
            _ew_mul(A1, A1, B1)
            pltpu.sync_copy(A1, x1.at[pl.ds((r0 + ib) * _CH, _CH)])
        return 0
    lax.fori_loop(0, _EC_FAST // 2, body, 0)


def _egather_kernel(X0b_pad, tails2, heads2):
    i32, bf16 = jnp.int32, jnp.bfloat16
    fn = pl.kernel(
        _egather_body,
        out_type=jax.ShapeDtypeStruct((_E_PAD, _F), jnp.float32),
        mesh=_sc_mesh(),
        scratch_types=[pltpu.VMEM((_EC_FAST, _CH), i32), pltpu.VMEM((_EC_FAST, _CH), i32),
                       pltpu.VMEM((_CH, _F), jnp.float32), pltpu.VMEM((_CH, _F), jnp.float32),
                       pltpu.VMEM((_CH, _F), jnp.float32), pltpu.VMEM((_CH, _F), jnp.float32),
                       pltpu.SemaphoreType.DMA, pltpu.SemaphoreType.DMA],
    )
    return fn(X0b_pad, tails2, heads2)


def _tgather_body(xb, v02, v12, v22, x2, V0, V1, V2,
                  A0, B0, C0, A1, B1, C1, s0, s1):
    c = lax.axis_index("c")
    s = lax.axis_index("s")
    slow = c == _SLOW_CORE
    nch = jnp.where(slow, _TC_SLOW, _TC_FAST)
    r0 = jnp.where(slow, s * _TC_SLOW, _NS * _TC_SLOW + s * _TC_FAST)
    pltpu.sync_copy(v02.at[pl.ds(r0, _TC_FAST)], V0)
    pltpu.sync_copy(v12.at[pl.ds(r0, _TC_FAST)], V1)
    pltpu.sync_copy(v22.at[pl.ds(r0, _TC_FAST)], V2)

    def body(j, _):
        @pl.when(2 * j + 1 < nch)
        def _():
            ia = 2 * j
            ib = 2 * j + 1
            da1 = pltpu.async_copy(xb.at[V0.at[ia]], A0, s0)
            da2 = pltpu.async_copy(xb.at[V1.at[ia]], B0, s0)
            da3 = pltpu.async_copy(xb.at[V2.at[ia]], C0, s0)
            db1 = pltpu.async_copy(xb.at[V0.at[ib]], A1, s1)
            db2 = pltpu.async_copy(xb.at[V1.at[ib]], B1, s1)
            db3 = pltpu.async_copy(xb.at[V2.at[ib]], C1, s1)
            da1.wait()
            da2.wait()
            da3.wait()
            _ew_mul(A0, A0, B0)
            _ew_mul(A0, A0, C0)
            pltpu.sync_copy(A0, x2.at[pl.ds((r0 + ia) * _CH, _CH)])
            db1.wait()
            db2.wait()
            db3.wait()
            _ew_mul(A1, A1, B1)
            _ew_mul(A1, A1, C1)
            pltpu.sync_copy(A1, x2.at[pl.ds((r0 + ib) * _CH, _CH)])
        return 0
    lax.fori_loop(0, _TC_FAST // 2, body, 0)


def _tgather_kernel(X0b_pad, v02, v12, v22):
    i32, bf16 = jnp.int32, jnp.bfloat16
    fn = pl.kernel(
        _tgather_body,
        out_type=jax.ShapeDtypeStruct((_T_PAD, _F), jnp.float32),
        mesh=_sc_mesh(),
        scratch_types=[pltpu.VMEM((_TC_FAST, _CH), i32)] * 3 +
                      [pltpu.VMEM((_CH, _F), jnp.float32)] * 6 +
                      [pltpu.SemaphoreType.DMA, pltpu.SemaphoreType.DMA],
    )
    return fn(X0b_pad, v02, v12, v22)


def _trans_body(tails, heads, t02, t12, t22,
                p0o, p1o, p2o, n0o, n1o, n2o,
                T0, T1, T2, P0, P1, P2, N0, N1, N2, sm):
    w = _wid()
    r0 = w * _TCH_W
    pltpu.sync_copy(t02.at[pl.ds(r0, _TCH_W)], T0)
    pltpu.sync_copy(t12.at[pl.ds(r0, _TCH_W)], T1)
    pltpu.sync_copy(t22.at[pl.ds(r0, _TCH_W)], T2)

    def body(i, _):
        ds = [pltpu.async_copy(heads.at[T0.at[i]], P0.at[i], sm),
              pltpu.async_copy(tails.at[T0.at[i]], N0.at[i], sm),
              pltpu.async_copy(tails.at[T1.at[i]], P1.at[i], sm),
              pltpu.async_copy(heads.at[T1.at[i]], N1.at[i], sm),
              pltpu.async_copy(heads.at[T2.at[i]], P2.at[i], sm),
              pltpu.async_copy(tails.at[T2.at[i]], N2.at[i], sm)]
        for d in ds:
            d.wait()
        return 0
    lax.fori_loop(0, _TCH_W, body, 0)

    pltpu.sync_copy(P0, p0o.at[pl.ds(r0, _TCH_W)])
    pltpu.sync_copy(P1, p1o.at[pl.ds(r0, _TCH_W)])
    pltpu.sync_copy(P2, p2o.at[pl.ds(r0, _TCH_W)])
    pltpu.sync_copy(N0, n0o.at[pl.ds(r0, _TCH_W)])
    pltpu.sync_copy(N1, n1o.at[pl.ds(r0, _TCH_W)])
    pltpu.sync_copy(N2, n2o.at[pl.ds(r0, _TCH_W)])


def _trans_kernel(tails, heads, t02, t12, t22):
    i32 = jnp.int32
    fn = pl.kernel(
        _trans_body,
        out_type=[jax.ShapeDtypeStruct((_T_PAD // _CH, _CH), i32)] * 6,
        mesh=_sc_mesh(),
        scratch_types=[pltpu.VMEM((_TCH_W, _CH), i32)] * 9 +
                      [pltpu.SemaphoreType.DMA],
    )
    return fn(tails, heads, t02, t12, t22)


def _and_body(xb, tails2, heads2, v02, v12, v22, x1, x2,
              IT, IH, V0, V1, V2, A, B, sm, xs):
    c = lax.axis_index("c")
    s = lax.axis_index("s")
    w = s * _NC + c
    rows_sub = _N_PAD // _NS
    off = s * rows_sub
    pltpu.sync_copy(xb.at[pl.ds(off, rows_sub)], xs.at[pl.ds(off, rows_sub)])
    plsc.subcore_barrier()

    r0 = w * _ECH_W

    for h in range(2):
        hr = r0 + h * (_ECH_W // 2)
        pltpu.sync_copy(tails2.at[pl.ds(hr, _ECH_W // 2)], IT)
        pltpu.sync_copy(heads2.at[pl.ds(hr, _ECH_W // 2)], IH)

        def ebody(k, _, hr=hr):
            d1 = pltpu.async_copy(xs.at[IT.at[k]], A, sm)
            d2 = pltpu.async_copy(xs.at[IH.at[k]], B, sm)
            d1.wait()
            d2.wait()
            _ew_mul(A, A, B)
            pltpu.sync_copy(A, x1.at[pl.ds((hr + k) * _CH, _CH)])
            return 0
        lax.fori_loop(0, _ECH_W // 2, ebody, 0)

    t0 = w * _TCH_W

    for h in range(5):
        hr = t0 + h * (_TCH_W // 5)
        pltpu.sync_copy(v02.at[pl.ds(hr, _TCH_W // 5)], V0)
        pltpu.sync_copy(v12.at[pl.ds(hr, _TCH_W // 5)], V1)
        pltpu.sync_copy(v22.at[pl.ds(hr, _TCH_W // 5)], V2)

        def tbody(k, _, hr=hr):
            d1 = pltpu.async_copy(xs.at[V0.at[k]], A, sm)
            d2 = pltpu.async_copy(xs.at[V1.at[k]], B, sm)
            d1.wait()
            d2.wait()
            _ew_mul(A, A, B)
            d3 = pltpu.async_copy(xs.at[V2.at[k]], B, sm)
            d3.wait()
            _ew_mul(A, A, B)
            pltpu.sync_copy(A, x2.at[pl.ds((hr + k) * _CH, _CH)])
            return 0
        lax.fori_loop(0, _TCH_W // 5, tbody, 0)


def _and_kernel(X0b_pad, tails2, heads2, v02, v12, v22):
    i32, f32 = jnp.int32, jnp.float32
    fn = pl.kernel(
        _and_body,
        out_type=[jax.ShapeDtypeStruct((_E_PAD, _F), f32),
                  jax.ShapeDtypeStruct((_T_PAD, _F), f32)],
        mesh=_sc_mesh(),
        scratch_types=[pltpu.VMEM((_ECH_W // 2, _CH), i32)] * 2 +
                      [pltpu.VMEM((_TCH_W // 5, _CH), i32)] * 3 +
                      [pltpu.VMEM((_CH, _F), f32)] * 2 +
                      [pltpu.SemaphoreType.DMA,
                       pltpu.VMEM_SHARED((_N_PAD, _F), f32)],
    )
    return fn(X0b_pad, tails2, heads2, v02, v12, v22)


_E_PER_W = _E_PAD // _NW
_T_PER_W = _T_PAD // _NW


def _gather_body_r1(xb, tails, heads, v0, v1, v2, t0, t1, t2,
                    x1, x2, p0, p1, p2, n0, n1, n2,
                    ia, ib, ic, A, B, C, sg, sem):
    w = _wid()
    ebase = w * _E_PER_W

    def edge_chunk(i, _):
        base = pl.multiple_of(ebase + i * _CH, _CH)
        pltpu.sync_copy(tails.at[pl.ds(base, _CH)], ia)
        pltpu.sync_copy(heads.at[pl.ds(base, _CH)], ib)
        pltpu.async_copy(xb.at[ia], A, sem).wait()
        pltpu.async_copy(xb.at[ib], B, sem).wait()
        _ew_mul(A, A, B)
        pltpu.sync_copy(A, x1.at[pl.ds(base, _CH)])
        return 0

    lax.fori_loop(0, _E_PER_W // _CH, edge_chunk, 0)

    tbase = w * _T_PER_W

    def tri_chunk(i, _):
        base = pl.multiple_of(tbase + i * _CH, _CH)
        pltpu.sync_copy(v0.at[pl.ds(base, _CH)], ia)
        pltpu.sync_copy(v1.at[pl.ds(base, _CH)], ib)
        pltpu.sync_copy(v2.at[pl.ds(base, _CH)], ic)
        pltpu.async_copy(xb.at[ia], A, sem).wait()
        pltpu.async_copy(xb.at[ib], B, sem).wait()
        pltpu.async_copy(xb.at[ic], C, sem).wait()
        _ew_mul(A, A, B)
        _ew_mul(A, A, C)
        pltpu.sync_copy(A, x2.at[pl.ds(base, _CH)])
        pltpu.sync_copy(t0.at[pl.ds(base, _CH)], ia)
        pltpu.async_copy(heads.at[ia], sg, sem).wait()
        pltpu.sync_copy(sg, p0.at[pl.ds(base, _CH)])
        pltpu.async_copy(tails.at[ia], sg, sem).wait()
        pltpu.sync_copy(sg, n0.at[pl.ds(base, _CH)])
        pltpu.sync_copy(t1.at[pl.ds(base, _CH)], ia)
        pltpu.async_copy(tails.at[ia], sg, sem).wait()
        pltpu.sync_copy(sg, p1.at[pl.ds(base, _CH)])
        pltpu.async_copy(heads.at[ia], sg, sem).wait()
        pltpu.sync_copy(sg, n1.at[pl.ds(base, _CH)])
        pltpu.sync_copy(t2.at[pl.ds(base, _CH)], ia)
        pltpu.async_copy(heads.at[ia], sg, sem).wait()
        pltpu.sync_copy(sg, p2.at[pl.ds(base, _CH)])
        pltpu.async_copy(tails.at[ia], sg, sem).wait()
        pltpu.sync_copy(sg, n2.at[pl.ds(base, _CH)])
        return 0

    lax.fori_loop(0, _T_PER_W // _CH, tri_chunk, 0)


def _gather_kernel_r1(X0b_pad, tails, heads, v0, v1, v2, t0, t1, t2):
    i32 = jnp.int32
    f32 = jnp.float32
    fn = pl.kernel(
        _gather_body_r1,
        out_type=[jax.ShapeDtypeStruct((_E_PAD, _F), f32),
                  jax.ShapeDtypeStruct((_T_PAD, _F), f32)] +
                 [jax.ShapeDtypeStruct((_T_PAD,), i32)] * 6,
        mesh=_sc_mesh(),
        scratch_types=[pltpu.VMEM((_CH,), i32), pltpu.VMEM((_CH,), i32),
                       pltpu.VMEM((_CH,), i32),
                       pltpu.VMEM((_CH, _F), f32), pltpu.VMEM((_CH, _F), f32),
                       pltpu.VMEM((_CH, _F), f32),
                       pltpu.VMEM((_CH,), i32),
                       pltpu.SemaphoreType.DMA],
    )
    return fn(X0b_pad, tails, heads, v0, v1, v2, t0, t1, t2)


def _scatter_body(h1, g2, tails1, heads1, p01, p11, p21, n01, n11, n21,
                  z, it, ih, q0, q1, q2, q3, q4, q5,
                  R, Rn, li, lr, acc):
    c = lax.axis_index("c")
    s = lax.axis_index("s")
    w = s * _NC + c
    rows_sub = _N_PAD // _NS
    zr0 = s * rows_sub

    def zfill(j, _):
        R[j // (_F // 16), pl.ds((j % (_F // 16)) * 16, 16)] = jnp.zeros((16,), jnp.float32)
        return 0
    lax.fori_loop(0, _CH * (_F // 16), zfill, 0)

    def zcp(k, _):
        off = zr0 + k * _CH
        pltpu.sync_copy(R, acc.at[pl.ds(off, _CH)])
        return 0
    lax.fori_loop(0, rows_sub // _CH, zcp, 0)

    plsc.subcore_barrier()

    ebase = w * _ECH_W

    def edge_body(k, _):
        base = (ebase + k) * _CH
        d1 = pltpu.async_copy(tails1.at[pl.ds(base, _CH)], it, li)
        d2 = pltpu.async_copy(heads1.at[pl.ds(base, _CH)], ih, li)
        d3 = pltpu.async_copy(h1.at[pl.ds(base, _CH)], R, lr)
        d1.wait()
        d2.wait()
        d3.wait()
        _ew_neg(Rn, R, _F)
        pltpu.sync_copy(R, acc.at[ih], add=True)
        pltpu.sync_copy(Rn, acc.at[it], add=True)
        return 0

    lax.fori_loop(0, _ECH_W, edge_body, 0)

    tbase = w * _TCH_W

    def tri_body(k, _):
        base = (tbase + k) * _CH
        dd = [pltpu.async_copy(p01.at[pl.ds(base, _CH)], q0, li),
              pltpu.async_copy(p11.at[pl.ds(base, _CH)], q1, li),
              pltpu.async_copy(p21.at[pl.ds(base, _CH)], q2, li),
              pltpu.async_copy(n01.at[pl.ds(base, _CH)], q3, li),
              pltpu.async_copy(n11.at[pl.ds(base, _CH)], q4, li),
              pltpu.async_copy(n21.at[pl.ds(base, _CH)], q5, li),
              pltpu.async_copy(g2.at[pl.ds(base, _CH)], R, lr)]
        for d in dd:
            d.wait()
        _ew_neg(Rn, R, _F)
        pltpu.sync_copy(R, acc.at[q0], add=True)
        pltpu.sync_copy(R, acc.at[q1], add=True)
        pltpu.sync_copy(R, acc.at[q2], add=True)
        pltpu.sync_copy(Rn, acc.at[q3], add=True)
        pltpu.sync_copy(Rn, acc.at[q4], add=True)
        pltpu.sync_copy(Rn, acc.at[q5], add=True)
        return 0

    lax.fori_loop(0, _TCH_W, tri_body, 0)

    plsc.subcore_barrier()

    def outcp(k, _):
        off = zr0 + k * _CH
        pltpu.sync_copy(acc.at[pl.ds(off, _CH)], z.at[c, pl.ds(off, _CH)])
        return 0
    lax.fori_loop(0, rows_sub // _CH, outcp, 0)


def _scatter_kernel(H1p, G2, tails1, heads1, p01, p11, p21, n01, n11, n21):
    i32, f32 = jnp.int32, jnp.float32
    fn = pl.kernel(
        _scatter_body,
        out_type=jax.ShapeDtypeStruct((_NC, _N_PAD, _F), f32),
        mesh=_sc_mesh(),
        scratch_types=[pltpu.VMEM((_CH,), i32)] * 8 +
                      [pltpu.VMEM((_CH, _F), f32)] * 2 +
                      [pltpu.SemaphoreType.DMA] * 2 +
                      [pltpu.VMEM_SHARED((_N_PAD, _F), f32)],
    )
    return fn(H1p, G2, tails1, heads1, p01, p11, p21, n01, n11, n21)


def kernel(X0, edge_index, tri_index, tri_edge_index, W_n, b_n, W_e, b_e,
           W_t, b_t, W_tri, b_tri, prelu_w):
    i32 = jnp.int32
    f32 = jnp.float32

    def padi2(col, total, fill):
        v = jnp.concatenate([col, jnp.full((total - col.shape[0],), fill, i32)])
        return v.reshape(total // _CH, _CH)

    tails2 = padi2(edge_index[:, 0], _E_PAD, _N)
    heads2 = padi2(edge_index[:, 1], _E_PAD, _N)
    v02 = padi2(tri_index[:, 0], _T_PAD, _N)
    v12 = padi2(tri_index[:, 1], _T_PAD, _N)
    v22 = padi2(tri_index[:, 2], _T_PAD, _N)
    t02 = padi2(tri_edge_index[:, 0], _T_PAD, _E)
    t12 = padi2(tri_edge_index[:, 1], _T_PAD, _E)
    t22 = padi2(tri_edge_index[:, 2], _T_PAD, _E)
    tails1 = tails2.reshape(_E_PAD)
    heads1 = heads2.reshape(_E_PAD)

    X0b, H0 = _node_kernel(X0, W_n, b_n, prelu_w)
    X0b_pad = jnp.concatenate([X0b, jnp.zeros((_N_PAD - _N, _F), f32)])

    X1, X2 = _and_kernel(X0b_pad, tails2, heads2, v02, v12, v22)
    p02, p12, p22, n02, n12, n22 = _trans_kernel(tails1, heads1, t02, t12, t22)

    H1p = _edge_mm_kernel(X1, W_e, b_e, b_tri, prelu_w)
    G2 = _tri_mm_kernel(X2, W_t, b_t, W_tri, prelu_w)

    Z = _scatter_kernel(H1p, G2, tails1, heads1,
                        p02.reshape(_T_PAD), p12.reshape(_T_PAD),
                        p22.reshape(_T_PAD), n02.reshape(_T_PAD),
                        n12.reshape(_T_PAD), n22.reshape(_T_PAD))

    return _combine_kernel(H0, Z[0, :_N], Z[1, :_N])
